# baseline (speedup 1.0000x reference)
import functools
import math

import jax
import jax.numpy as jnp
from jax import lax
from jax.experimental import pallas as pl
from jax.experimental.pallas import tpu as pltpu
from jax.experimental.pallas import tpu_sc as plsc

EMB = 128
SCALE = math.sqrt(float(EMB))
NC = 2
NS = 16
NW = NC * NS
LANES = 16
CHUNK = 128
NBUF = 2


@functools.partial(jax.jit, static_argnums=(2,))
def _embed(tokens_grp, table, vocab):
    nchunks = tokens_grp.shape[1]
    bpw = nchunks * CHUNK
    b = NW * bpw
    mesh = plsc.VectorSubcoreMesh(core_axis_name="c", subcore_axis_name="s")

    @functools.partial(
        pl.kernel,
        mesh=mesh,
        out_type=jax.ShapeDtypeStruct((b, EMB), jnp.float32),
        scratch_types=[
            pltpu.VMEM((nchunks, CHUNK), jnp.int32),
            pltpu.VMEM((NBUF, CHUNK, EMB), jnp.float32),
            pltpu.VMEM((NBUF, CHUNK, EMB), jnp.float32),
            pltpu.SemaphoreType.DMA,
            pltpu.SemaphoreType.DMA,
            pltpu.SemaphoreType.DMA,
            pltpu.SemaphoreType.DMA,
        ],
    )
    def k(tok_hbm, table_hbm, out_hbm, idx_v, gbuf, sbuf, sg0, sg1, ss0, ss1):
        wid = lax.axis_index("s") * NC + lax.axis_index("c")
        base = wid * bpw
        sg = (sg0, sg1)
        ss = (ss0, ss1)

        pltpu.sync_copy(tok_hbm.at[wid], idx_v)

        def clamp_body(r, c):
            for j in range(CHUNK // LANES):
                s = pl.ds(j * LANES, LANES)
                idx_v[r, s] = jnp.minimum(idx_v[r, s], vocab - 1)
            return c

        lax.fori_loop(0, nchunks, clamp_body, 0, unroll=2)

        for p in range(NBUF):
            pltpu.async_copy(table_hbm.at[idx_v.at[p]], gbuf.at[p], sg[p])

        def turn(g, slot):
            pltpu.make_async_copy(
                table_hbm.at[idx_v.at[g]], gbuf.at[slot], sg[slot]
            ).wait()
            @pl.when(g >= NBUF)
            def _():
                pltpu.make_async_copy(
                    sbuf.at[slot],
                    out_hbm.at[pl.ds(base + (g - NBUF) * CHUNK, CHUNK)],
                    ss[slot],
                ).wait()

            def scale_body(r, c):
                for j in range(EMB // LANES):
                    s = pl.ds(j * LANES, LANES)
                    sbuf[slot, r, s] = gbuf[slot, r, s] * SCALE
                return c

            lax.fori_loop(0, CHUNK, scale_body, 0, unroll=2)
            pltpu.async_copy(
                sbuf.at[slot],
                out_hbm.at[pl.ds(base + g * CHUNK, CHUNK)],
                ss[slot],
            )
            @pl.when(g + NBUF < nchunks)
            def _():
                pltpu.async_copy(
                    table_hbm.at[idx_v.at[g + NBUF]], gbuf.at[slot], sg[slot]
                )

        def round_body(i, c):
            for slot in range(NBUF):
                turn(i * NBUF + slot, slot)
            return c

        lax.fori_loop(0, nchunks // NBUF, round_body, 0)

        for p in range(NBUF):
            g = nchunks - NBUF + p
            pltpu.make_async_copy(
                sbuf.at[p % NBUF],
                out_hbm.at[pl.ds(base + g * CHUNK, CHUNK)],
                ss[p % NBUF],
            ).wait()

    return k(tokens_grp, table)


def kernel(tokens, table):
    b0, b1 = tokens.shape
    b = b0 * b1
    tokens_grp = tokens.reshape(NW, b // (NW * CHUNK), CHUNK)
    out = _embed(tokens_grp, table, table.shape[0])
    return out.reshape(b0, b1, EMB)
